# baseline (speedup 1.0000x reference)
import functools

import jax
import jax.numpy as jnp
from jax import lax
from jax.experimental import pallas as pl
from jax.experimental.pallas import tpu as pltpu
from jax.experimental.pallas import tpu_sc as plsc

_B, _R, _C = 2, 8192, 4096
_ROWS_PER_W = (_B * _R) // 32
_CHUNK = 8
_STEPS = _ROWS_PER_W // _CHUNK

_mesh = plsc.VectorSubcoreMesh(core_axis_name="c", subcore_axis_name="s")


@functools.partial(
    pl.kernel,
    out_type=jax.ShapeDtypeStruct((_B, _R, _C), jnp.float32),
    mesh=_mesh,
    scratch_types=[
        pltpu.VMEM((2, _CHUNK, _C), jnp.float32),
        pltpu.SemaphoreType.DMA,
        pltpu.SemaphoreType.DMA,
        pltpu.SemaphoreType.DMA,
        pltpu.SemaphoreType.DMA,
    ],
)
def _copy_fill(x_ref, o_ref, bufs, li0, li1, lo0, lo1):
    wid = lax.axis_index("s") * 2 + lax.axis_index("c")
    base = wid * _ROWS_PER_W
    b = base // _R
    r0 = base % _R

    lsem = (li0, li1)
    ssem = (lo0, lo1)

    loads = [None, None]
    stores = [None, None]

    loads[0] = pltpu.async_copy(
        x_ref.at[b, pl.ds(r0, _CHUNK), :], bufs.at[0], lsem[0])
    for g in range(_STEPS):
        cur = g % 2
        nxt = 1 - cur
        if g + 1 < _STEPS:
            if stores[nxt] is not None:
                stores[nxt].wait()
            loads[nxt] = pltpu.async_copy(
                x_ref.at[b, pl.ds(r0 + _CHUNK * (g + 1), _CHUNK), :],
                bufs.at[nxt], lsem[nxt])
        loads[cur].wait()
        stores[cur] = pltpu.async_copy(
            bufs.at[cur], o_ref.at[b, pl.ds(r0 + _CHUNK * g, _CHUNK), :],
            ssem[cur])
    stores[0].wait()
    stores[1].wait()

    @pl.when(r0 == 0)
    def _():
        def fill_row(i, _):
            bufs[0, 0, pl.ds(16 * i, 16)] = jnp.full((16,), -1.0, jnp.float32)
            return 0
        lax.fori_loop(0, _C // 16, fill_row, 0)
        for r in range(3):
            pltpu.async_copy(bufs.at[0, 0, :], o_ref.at[b, r, :], li0).wait()


def kernel(x):
    return _copy_fill(x)


# speedup vs baseline: 1.2459x; 1.2459x over previous
import jax
import jax.numpy as jnp
from jax.experimental import pallas as pl
from jax.experimental.pallas import tpu as pltpu

_B, _R, _C = 2, 8192, 4096
_CH = 128
_NBUF = 8
_NCHUNK = (_B * _R) // _CH
_PER_BATCH = _R // _CH


def _body(x_ref, o_ref, bufs, *sems):
    lsem = sems[:_NBUF]
    ssem = sems[_NBUF:]
    loads = [None] * _NBUF
    stores = [None] * _NBUF

    def src(c):
        return x_ref.at[c // _PER_BATCH, pl.ds((c % _PER_BATCH) * _CH, _CH), :]

    def dst(c):
        return o_ref.at[c // _PER_BATCH, pl.ds((c % _PER_BATCH) * _CH, _CH), :]

    ahead = _NBUF // 2
    for c in range(ahead):
        loads[c] = pltpu.make_async_copy(src(c), bufs.at[c], lsem[c])
        loads[c].start()
    for c in range(_NCHUNK):
        s = c % _NBUF
        loads[s].wait()
        if c % _PER_BATCH == 0:
            bufs[s, 0:3, :] = jnp.full((3, _C), -1.0, jnp.float32)
        stores[s] = pltpu.make_async_copy(bufs.at[s], dst(c), ssem[s])
        stores[s].start()
        nxt = c + ahead
        if nxt < _NCHUNK:
            t = nxt % _NBUF
            if stores[t] is not None:
                stores[t].wait()
            loads[t] = pltpu.make_async_copy(src(nxt), bufs.at[t], lsem[t])
            loads[t].start()
    for s in range(_NBUF):
        stores[s].wait()


def kernel(x):
    return pl.pallas_call(
        _body,
        in_specs=[pl.BlockSpec(memory_space=pl.ANY)],
        out_specs=pl.BlockSpec(memory_space=pl.ANY),
        out_shape=jax.ShapeDtypeStruct(x.shape, x.dtype),
        scratch_shapes=(
            [pltpu.VMEM((_NBUF, _CH, _C), jnp.float32)]
            + [pltpu.SemaphoreType.DMA] * (2 * _NBUF)
        ),
    )(x)


# speedup vs baseline: 1.2491x; 1.0026x over previous
import jax
import jax.numpy as jnp
from jax.experimental import pallas as pl
from jax.experimental.pallas import tpu as pltpu

_B, _R, _C = 2, 8192, 4096
_CH = 512
_NBUF = 6
_NCHUNK = (_B * _R) // _CH
_PER_BATCH = _R // _CH


def _body(x_ref, o_ref, bufs, *sems):
    lsem = sems[:_NBUF]
    ssem = sems[_NBUF:]
    loads = [None] * _NBUF
    stores = [None] * _NBUF

    def src(c):
        return x_ref.at[c // _PER_BATCH, pl.ds((c % _PER_BATCH) * _CH, _CH), :]

    def dst(c):
        return o_ref.at[c // _PER_BATCH, pl.ds((c % _PER_BATCH) * _CH, _CH), :]

    ahead = _NBUF // 2
    for c in range(ahead):
        loads[c] = pltpu.make_async_copy(src(c), bufs.at[c], lsem[c])
        loads[c].start()
    for c in range(_NCHUNK):
        s = c % _NBUF
        loads[s].wait()
        if c % _PER_BATCH == 0:
            bufs[s, 0:3, :] = jnp.full((3, _C), -1.0, jnp.float32)
        stores[s] = pltpu.make_async_copy(bufs.at[s], dst(c), ssem[s])
        stores[s].start()
        nxt = c + ahead
        if nxt < _NCHUNK:
            t = nxt % _NBUF
            if stores[t] is not None:
                stores[t].wait()
            loads[t] = pltpu.make_async_copy(src(nxt), bufs.at[t], lsem[t])
            loads[t].start()
    for s in range(_NBUF):
        stores[s].wait()


def kernel(x):
    return pl.pallas_call(
        _body,
        in_specs=[pl.BlockSpec(memory_space=pl.ANY)],
        out_specs=pl.BlockSpec(memory_space=pl.ANY),
        out_shape=jax.ShapeDtypeStruct(x.shape, x.dtype),
        scratch_shapes=(
            [pltpu.VMEM((_NBUF, _CH, _C), jnp.float32)]
            + [pltpu.SemaphoreType.DMA] * (2 * _NBUF)
        ),
    )(x)


# speedup vs baseline: 1.2544x; 1.0042x over previous
import jax
import jax.numpy as jnp
from jax import lax
from jax.experimental import pallas as pl
from jax.experimental.pallas import tpu as pltpu

_BLK = 960


def _body(x_ref, o_ref):
    j = pl.program_id(1)

    @pl.when(j == 0)
    def _():
        v = x_ref[...]
        row = lax.broadcasted_iota(jnp.int32, v.shape, 1)
        o_ref[...] = jnp.where(row < 3, jnp.float32(-1.0), v)

    @pl.when(j != 0)
    def _():
        o_ref[...] = x_ref[...]


def kernel(x):
    b, r, c = x.shape
    return pl.pallas_call(
        _body,
        grid=(b, pl.cdiv(r, _BLK)),
        in_specs=[pl.BlockSpec((1, _BLK, c), lambda i, j: (i, j, 0))],
        out_specs=pl.BlockSpec((1, _BLK, c), lambda i, j: (i, j, 0)),
        out_shape=jax.ShapeDtypeStruct(x.shape, x.dtype),
        compiler_params=pltpu.CompilerParams(vmem_limit_bytes=100 * 1024 * 1024),
    )(x)
